# baseline (speedup 1.0000x reference)
import functools

import jax
import jax.numpy as jnp
from jax.experimental import pallas as pl
from jax.experimental.pallas import tpu as pltpu

BLOCK_M = 16384
PACK = 8


def _body(x_ref, w_ref, b_ref, o_ref):
    o = (
        jnp.dot(x_ref[...], w_ref[...], preferred_element_type=jnp.float32)
        + b_ref[...]
    )
    o3 = o.reshape(o.shape[0] // PACK, PACK, o.shape[1])
    o_ref[...] = jnp.concatenate(
        [o3[:, v, :] for v in range(PACK)], axis=1
    )


@functools.partial(jax.jit, static_argnames=())
def kernel(features, indices, W, b):
    del indices
    m, c_in = features.shape
    c_out = W.shape[1]
    block_m = min(BLOCK_M, m)
    n_blocks = pl.cdiv(m, block_m)
    packed = pl.pallas_call(
        _body,
        grid=(n_blocks,),
        in_specs=[
            pl.BlockSpec((block_m, c_in), lambda i: (i, 0)),
            pl.BlockSpec((c_in, c_out), lambda i: (0, 0)),
            pl.BlockSpec((1, c_out), lambda i: (0, 0)),
        ],
        out_specs=pl.BlockSpec((block_m // PACK, PACK * c_out), lambda i: (i, 0)),
        out_shape=jax.ShapeDtypeStruct((m // PACK, PACK * c_out), jnp.float32),
    )(features, W, b.reshape(1, c_out))
    return packed.reshape(m, c_out)


# speedup vs baseline: 5.3986x; 5.3986x over previous
import functools

import jax
import jax.numpy as jnp
from jax.experimental import pallas as pl

BLOCK_M = 32768


def _body(x_ref, w_ref, b_ref, o_ref):
    t = jax.lax.dot_general(
        w_ref[...], x_ref[...],
        dimension_numbers=(((0,), (1,)), ((), ())),
        preferred_element_type=jnp.float32,
    )
    o_ref[...] = t + b_ref[...]


@functools.partial(jax.jit, static_argnames=())
def kernel(features, indices, W, b):
    del indices
    m, c_in = features.shape
    c_out = W.shape[1]
    block_m = min(BLOCK_M, m)
    grid = (pl.cdiv(m, block_m),)
    out_t = pl.pallas_call(
        _body,
        grid=grid,
        in_specs=[
            pl.BlockSpec((block_m, c_in), lambda i: (i, 0)),
            pl.BlockSpec((c_in, c_out), lambda i: (0, 0)),
            pl.BlockSpec((c_out, 1), lambda i: (0, 0)),
        ],
        out_specs=pl.BlockSpec((c_out, block_m), lambda i: (0, i)),
        out_shape=jax.ShapeDtypeStruct((c_out, m), jnp.float32),
    )(features, W, b.reshape(c_out, 1))
    return out_t.T
